# Initial kernel scaffold; baseline (speedup 1.0000x reference)
import jax
import jax.numpy as jnp
from jax.experimental import pallas as pl


def kernel(x, edge_index, batch, W0, b0, gamma0, beta0, W1, b1, gamma1, beta1, W2, b2, gamma2, beta2, W_lin, b_lin):
    raise NotImplementedError("write your pallas kernel here")


# speedup vs baseline: 5.0294x; 5.0294x over previous
import functools

import jax
import jax.numpy as jnp
from jax import lax
from jax.experimental import pallas as pl
from jax.experimental.pallas import tpu as pltpu
from jax.experimental.pallas import tpu_sc as plsc

N = 10000
E = 160000
D_IN = 256
H = 256
O = 64
G = 64

NC = 2
NS = 16
L = 16

NPAD = 10240
EPAD = 163840
EB = 128
NB_SC = EPAD // (NS * EB)
NB_DEG = EPAD // (NC * NS * EB)
ROWS_PER_TILE = NPAD // NS

@functools.lru_cache(maxsize=None)
def _sc_mesh():
    return plsc.VectorSubcoreMesh(core_axis_name="c", subcore_axis_name="s",
                                  num_cores=NC, num_subcores=NS)


def _zero_block(buf, nrow, ncol):
    zeros = jnp.zeros((L,), jnp.float32)

    @pl.loop(0, nrow)
    def _(i):
        for j in range(ncol // L):
            buf[i, pl.ds(j * L, L)] = zeros


@functools.lru_cache(maxsize=None)
def _deg_kernel_fn():
    return pl.kernel(
        _deg_body,
        out_type=jax.ShapeDtypeStruct((NC, NPAD, L), jnp.float32),
        mesh=_sc_mesh(),
        scratch_types=[
            pltpu.VMEM((NB_DEG, EB), jnp.int32),
            pltpu.VMEM((EB, L), jnp.float32),
            pltpu.VMEM_SHARED((NPAD, L), jnp.float32),
        ],
    )


def _deg_body(dst_hbm, out_hbm, idx_v, buf_v, degw):
    c = lax.axis_index("c")
    s = lax.axis_index("s")
    wid = s * NC + c

    pltpu.sync_copy(dst_hbm.at[wid], idx_v)

    _zero_block(buf_v, EB, L)
    row0 = s * ROWS_PER_TILE
    for r in range(ROWS_PER_TILE // EB):
        pltpu.sync_copy(buf_v, degw.at[pl.ds(row0 + r * EB, EB)])

    ones = jnp.ones((L,), jnp.float32)

    @pl.loop(0, EB)
    def _(i):
        buf_v[i, :] = ones

    plsc.subcore_barrier()

    @pl.loop(0, NB_DEG)
    def _(j):
        pltpu.sync_copy(buf_v, degw.at[idx_v.at[j]], add=True)

    plsc.subcore_barrier()
    pltpu.sync_copy(degw.at[pl.ds(row0, ROWS_PER_TILE)],
                    out_hbm.at[c, pl.ds(row0, ROWS_PER_TILE)])


@functools.lru_cache(maxsize=None)
def _scatter_kernel_fn():
    return pl.kernel(
        _scatter_body,
        out_type=jax.ShapeDtypeStruct((NC, NPAD, 128), jnp.float32),
        mesh=_sc_mesh(),
        scratch_types=[
            pltpu.VMEM((NB_SC, EB), jnp.int32),
            pltpu.VMEM((NB_SC, EB), jnp.int32),
            pltpu.VMEM((EB, 128), jnp.float32),
            pltpu.VMEM_SHARED((NPAD, 128), jnp.float32),
            pltpu.SemaphoreType.DMA,
        ],
    )


def _scatter_body(hn_hbm, src_hbm, dst_hbm, out_hbm,
                  src_v, dst_v, rows_a, agg, sem_a):
    c = lax.axis_index("c")
    s = lax.axis_index("s")

    pltpu.sync_copy(src_hbm.at[c * NS + s], src_v)
    pltpu.sync_copy(dst_hbm.at[s], dst_v)

    _zero_block(rows_a, EB, 128)
    row0 = s * ROWS_PER_TILE
    for r in range(ROWS_PER_TILE // EB):
        pltpu.sync_copy(rows_a, agg.at[pl.ds(row0 + r * EB, EB)])
    plsc.subcore_barrier()

    @pl.loop(0, NB_SC)
    def _(j):
        pltpu.async_copy(hn_hbm.at[src_v.at[j]], rows_a, sem_a).wait()
        pltpu.sync_copy(rows_a, agg.at[dst_v.at[j]], add=True)

    plsc.subcore_barrier()
    pltpu.sync_copy(agg.at[pl.ds(row0, ROWS_PER_TILE)],
                    out_hbm.at[c, pl.ds(row0, ROWS_PER_TILE)])


def _mm0_body(x_ref, degw_ref, w_ref, hn_ref, d_ref):
    deg = degw_ref[0, :N, 0:1] + degw_ref[1, :N, 0:1] + 1.0
    d = lax.rsqrt(deg)
    d_ref[...] = d
    h = jnp.dot(x_ref[...], w_ref[...], preferred_element_type=jnp.float32)
    hn_ref[...] = h * d


def _post_conv(s_ref, hn_ref, d_ref, b_ref, g_ref, be_ref):
    sc = jnp.concatenate([s_ref[0, :N, :], s_ref[1, :N, :]], axis=1)
    z = (sc + hn_ref[...]) * d_ref[...] + b_ref[...]
    mean = jnp.mean(z, axis=0, keepdims=True)
    var = jnp.mean((z - mean) ** 2, axis=0, keepdims=True)
    zn = (z - mean) * lax.rsqrt(var + 1e-5) * g_ref[...] + be_ref[...]
    return jnp.maximum(zn, 0.0)


def _mm_mid_body(s_ref, hn_ref, d_ref, b_ref, g_ref, be_ref, w_ref, out_ref):
    a = _post_conv(s_ref, hn_ref, d_ref, b_ref, g_ref, be_ref)
    h = jnp.dot(a, w_ref[...], preferred_element_type=jnp.float32)
    out_ref[...] = h * d_ref[...]


def _mm_fin_body(s_ref, hn_ref, d_ref, b_ref, g_ref, be_ref, batch_ref,
                 wl_ref, bl_ref, out_ref):
    a = _post_conv(s_ref, hn_ref, d_ref, b_ref, g_ref, be_ref)
    gids = lax.broadcasted_iota(jnp.int32, (G, N), 0)
    oh_t = (gids == batch_ref[...]).astype(jnp.float32)
    sums = jnp.dot(oh_t, a, preferred_element_type=jnp.float32)
    counts = jnp.sum(oh_t, axis=1, keepdims=True)
    pooled = sums / jnp.maximum(counts, 1.0)
    out_ref[...] = jnp.dot(pooled, wl_ref[...],
                           preferred_element_type=jnp.float32) + bl_ref[...]


_f32 = jnp.float32

_mm0 = pl.pallas_call(
    _mm0_body,
    out_shape=[jax.ShapeDtypeStruct((N, H), _f32),
               jax.ShapeDtypeStruct((N, 1), _f32)],
)

_mm_mid = pl.pallas_call(
    _mm_mid_body,
    out_shape=jax.ShapeDtypeStruct((N, H), _f32),
)

_mm_fin = pl.pallas_call(
    _mm_fin_body,
    out_shape=jax.ShapeDtypeStruct((G, O), _f32),
)


def kernel(x, edge_index, batch, W0, b0, gamma0, beta0, W1, b1, gamma1, beta1,
           W2, b2, gamma2, beta2, W_lin, b_lin):
    src = edge_index[0]
    dst = edge_index[1]
    pad = EPAD - E
    srcp = jnp.concatenate([src, jnp.zeros((pad,), jnp.int32)])
    dstp = jnp.concatenate([dst, jnp.full((pad,), N, jnp.int32)])

    srclo = (srcp * 2).reshape(NS, NB_SC, EB)
    srchi = (srcp * 2 + 1).reshape(NS, NB_SC, EB)
    src_all = jnp.concatenate([srclo, srchi]).reshape(NC * NS, NB_SC, EB)
    dst_sc = dstp.reshape(NS, NB_SC, EB)
    dst_deg = dstp.reshape(NC * NS, NB_DEG, EB)

    degw = _deg_kernel_fn()(dst_deg)
    hn, d = _mm0(x, degw, W0)

    def layer_scatter(hn_k):
        return _scatter_kernel_fn()(hn_k.reshape(2 * N, 128), src_all, dst_sc)

    s0 = layer_scatter(hn)
    hn1 = _mm_mid(s0, hn, d, b0.reshape(1, H), gamma0.reshape(1, H),
                  beta0.reshape(1, H), W1)
    s1 = layer_scatter(hn1)
    hn2 = _mm_mid(s1, hn1, d, b1.reshape(1, H), gamma1.reshape(1, H),
                  beta1.reshape(1, H), W2)
    s2 = layer_scatter(hn2)
    return _mm_fin(s2, hn2, d, b2.reshape(1, H), gamma2.reshape(1, H),
                   beta2.reshape(1, H), batch.reshape(1, N), W_lin,
                   b_lin.reshape(1, O))


# speedup vs baseline: 5.9806x; 1.1891x over previous
import functools

import jax
import jax.numpy as jnp
from jax import lax
from jax.experimental import pallas as pl
from jax.experimental.pallas import tpu as pltpu
from jax.experimental.pallas import tpu_sc as plsc

N = 10000
E = 160000
D_IN = 256
H = 256
O = 64
G = 64

NC = 2
NS = 16
L = 16

NPAD = 10240
EPAD = 163840
EB = 128
CHB = 16
NB_SC = EPAD // (NS * EB)
NB_DEG = EPAD // (NC * NS * EB)
ROWS_PER_TILE = NPAD // NS

@functools.lru_cache(maxsize=None)
def _sc_mesh():
    return plsc.VectorSubcoreMesh(core_axis_name="c", subcore_axis_name="s",
                                  num_cores=NC, num_subcores=NS)


def _zero_block(buf, nrow, ncol):
    zeros = jnp.zeros((L,), jnp.float32)

    @pl.loop(0, nrow)
    def _(i):
        for j in range(ncol // L):
            buf[i, pl.ds(j * L, L)] = zeros


@functools.lru_cache(maxsize=None)
def _deg_kernel_fn():
    return pl.kernel(
        _deg_body,
        out_type=jax.ShapeDtypeStruct((NC, NPAD, L), jnp.float32),
        mesh=_sc_mesh(),
        scratch_types=[
            pltpu.VMEM((NB_DEG, EB), jnp.int32),
            pltpu.VMEM((EB, L), jnp.float32),
            pltpu.VMEM_SHARED((NPAD, L), jnp.float32),
        ],
    )


def _deg_body(dst_hbm, out_hbm, idx_v, buf_v, degw):
    c = lax.axis_index("c")
    s = lax.axis_index("s")
    wid = s * NC + c

    pltpu.sync_copy(dst_hbm.at[wid], idx_v)

    _zero_block(buf_v, EB, L)
    row0 = s * ROWS_PER_TILE
    for r in range(ROWS_PER_TILE // EB):
        pltpu.sync_copy(buf_v, degw.at[pl.ds(row0 + r * EB, EB)])

    ones = jnp.ones((L,), jnp.float32)

    @pl.loop(0, EB)
    def _(i):
        buf_v[i, :] = ones

    plsc.subcore_barrier()

    @pl.loop(0, NB_DEG)
    def _(j):
        pltpu.sync_copy(buf_v, degw.at[idx_v.at[j]], add=True)

    plsc.subcore_barrier()
    pltpu.sync_copy(degw.at[pl.ds(row0, ROWS_PER_TILE)],
                    out_hbm.at[c, pl.ds(row0, ROWS_PER_TILE)])


@functools.lru_cache(maxsize=None)
def _scatter_kernel_fn():
    return pl.kernel(
        _scatter_body,
        out_type=jax.ShapeDtypeStruct((NC, NPAD, 128), jnp.float32),
        mesh=_sc_mesh(),
        scratch_types=[
            pltpu.VMEM((NB_SC, EB), jnp.int32),
            pltpu.VMEM((2, CHB, EB), jnp.int32),
            pltpu.VMEM((EB, 128), jnp.float32),
            pltpu.VMEM((EB, 128), jnp.float32),
            pltpu.VMEM_SHARED((NPAD, 128), jnp.float32),
            pltpu.SemaphoreType.DMA,
            pltpu.SemaphoreType.DMA,
            pltpu.SemaphoreType.DMA,
        ],
    )


def _scatter_body(hn_hbm, src_hbm, dst_hbm, out_hbm,
                  src_v, dst_ch, rows_a, rows_b, agg, sem_a, sem_b, sem_i):
    c = lax.axis_index("c")
    s = lax.axis_index("s")

    pltpu.sync_copy(src_hbm.at[c * NS + s], src_v)
    pltpu.sync_copy(dst_hbm.at[s, pl.ds(0, CHB)], dst_ch.at[0])

    _zero_block(rows_a, EB, 128)
    row0 = s * ROWS_PER_TILE
    for r in range(ROWS_PER_TILE // EB):
        pltpu.sync_copy(rows_a, agg.at[pl.ds(row0 + r * EB, EB)])
    plsc.subcore_barrier()

    pltpu.async_copy(hn_hbm.at[src_v.at[0]], rows_a, sem_a)
    for ch in range(NB_SC // CHB):
        k = ch % 2
        base = ch * CHB
        if ch > 0:
            pltpu.make_async_copy(dst_hbm.at[s, pl.ds(base, CHB)],
                                  dst_ch.at[k], sem_i).wait()
        if ch + 1 < NB_SC // CHB:
            pltpu.async_copy(dst_hbm.at[s, pl.ds(base + CHB, CHB)],
                             dst_ch.at[(ch + 1) % 2], sem_i)

        @pl.loop(0, CHB // 2)
        def _(jj):
            j = base + jj * 2
            pltpu.async_copy(hn_hbm.at[src_v.at[j + 1]], rows_b, sem_b)
            pltpu.make_async_copy(hn_hbm.at[src_v.at[j]], rows_a, sem_a).wait()
            pltpu.sync_copy(rows_a, agg.at[dst_ch.at[k, jj * 2]], add=True)

            @pl.when(j + 2 < NB_SC)
            def _():
                pltpu.async_copy(hn_hbm.at[src_v.at[j + 2]], rows_a, sem_a)

            pltpu.make_async_copy(hn_hbm.at[src_v.at[j + 1]], rows_b,
                                  sem_b).wait()
            pltpu.sync_copy(rows_b, agg.at[dst_ch.at[k, jj * 2 + 1]], add=True)

    plsc.subcore_barrier()
    pltpu.sync_copy(agg.at[pl.ds(row0, ROWS_PER_TILE)],
                    out_hbm.at[c, pl.ds(row0, ROWS_PER_TILE)])


def _mm0_body(x_ref, degw_ref, w_ref, hn_ref, d_ref):
    deg = degw_ref[0, :N, 0:1] + degw_ref[1, :N, 0:1] + 1.0
    d = lax.rsqrt(deg)
    d_ref[...] = d
    h = jnp.dot(x_ref[...], w_ref[...], preferred_element_type=jnp.float32)
    hn_ref[...] = h * d


def _post_conv(s_ref, hn_ref, d_ref, b_ref, g_ref, be_ref):
    sc = jnp.concatenate([s_ref[0, :N, :], s_ref[1, :N, :]], axis=1)
    z = (sc + hn_ref[...]) * d_ref[...] + b_ref[...]
    mean = jnp.mean(z, axis=0, keepdims=True)
    var = jnp.mean((z - mean) ** 2, axis=0, keepdims=True)
    zn = (z - mean) * lax.rsqrt(var + 1e-5) * g_ref[...] + be_ref[...]
    return jnp.maximum(zn, 0.0)


def _mm_mid_body(s_ref, hn_ref, d_ref, b_ref, g_ref, be_ref, w_ref, out_ref):
    a = _post_conv(s_ref, hn_ref, d_ref, b_ref, g_ref, be_ref)
    h = jnp.dot(a, w_ref[...], preferred_element_type=jnp.float32)
    out_ref[...] = h * d_ref[...]


def _mm_fin_body(s_ref, hn_ref, d_ref, b_ref, g_ref, be_ref, batch_ref,
                 wl_ref, bl_ref, out_ref):
    a = _post_conv(s_ref, hn_ref, d_ref, b_ref, g_ref, be_ref)
    gids = lax.broadcasted_iota(jnp.int32, (G, N), 0)
    oh_t = (gids == batch_ref[...]).astype(jnp.float32)
    sums = jnp.dot(oh_t, a, preferred_element_type=jnp.float32)
    counts = jnp.sum(oh_t, axis=1, keepdims=True)
    pooled = sums / jnp.maximum(counts, 1.0)
    out_ref[...] = jnp.dot(pooled, wl_ref[...],
                           preferred_element_type=jnp.float32) + bl_ref[...]


_f32 = jnp.float32

_mm0 = pl.pallas_call(
    _mm0_body,
    out_shape=[jax.ShapeDtypeStruct((N, H), _f32),
               jax.ShapeDtypeStruct((N, 1), _f32)],
)

_mm_mid = pl.pallas_call(
    _mm_mid_body,
    out_shape=jax.ShapeDtypeStruct((N, H), _f32),
)

_mm_fin = pl.pallas_call(
    _mm_fin_body,
    out_shape=jax.ShapeDtypeStruct((G, O), _f32),
)


def kernel(x, edge_index, batch, W0, b0, gamma0, beta0, W1, b1, gamma1, beta1,
           W2, b2, gamma2, beta2, W_lin, b_lin):
    src = edge_index[0]
    dst = edge_index[1]
    pad = EPAD - E
    srcp = jnp.concatenate([src, jnp.zeros((pad,), jnp.int32)])
    dstp = jnp.concatenate([dst, jnp.full((pad,), N, jnp.int32)])

    srclo = (srcp * 2).reshape(NS, NB_SC, EB)
    srchi = (srcp * 2 + 1).reshape(NS, NB_SC, EB)
    src_all = jnp.concatenate([srclo, srchi]).reshape(NC * NS, NB_SC, EB)
    dst_sc = dstp.reshape(NS, NB_SC, EB)
    dst_deg = dstp.reshape(NC * NS, NB_DEG, EB)

    degw = _deg_kernel_fn()(dst_deg)
    hn, d = _mm0(x, degw, W0)

    def layer_scatter(hn_k):
        return _scatter_kernel_fn()(hn_k.reshape(2 * N, 128), src_all, dst_sc)

    s0 = layer_scatter(hn)
    hn1 = _mm_mid(s0, hn, d, b0.reshape(1, H), gamma0.reshape(1, H),
                  beta0.reshape(1, H), W1)
    s1 = layer_scatter(hn1)
    hn2 = _mm_mid(s1, hn1, d, b1.reshape(1, H), gamma1.reshape(1, H),
                  beta1.reshape(1, H), W2)
    s2 = layer_scatter(hn2)
    return _mm_fin(s2, hn2, d, b2.reshape(1, H), gamma2.reshape(1, H),
                   beta2.reshape(1, H), batch.reshape(1, N), W_lin,
                   b_lin.reshape(1, O))
